# baseline (device time: 16629 ns/iter reference)
import jax
import jax.numpy as jnp
from jax import lax
from jax.experimental import pallas as pl
from jax.experimental.pallas import tpu as pltpu

N_DEV = 4
B, Sq, Skv, Hq, Dh = 2, 256, 1024, 4, 64
HD = Hq * Dh
D = 512
S_SH = Skv // N_DEV
NCH = 2
SQ_C = Sq // NCH
NC = B * NCH
RC = SQ_C + 8
F32 = jnp.float32
BF16 = jnp.bfloat16


def kernel(x, Wq, K_ext, V_ext, Wo):

    def body(x_ref, wq_ref, k_ref, v_ref, wo_ref, out_ref,
             pbuf, rbuf, csend, crecv):
        my = lax.axis_index("i")
        left = lax.rem(my + N_DEV - 1, N_DEV)
        right = lax.rem(my + 1, N_DEV)
        diag = lax.rem(my + 2, N_DEV)

        barrier = pltpu.get_barrier_semaphore()
        for nbr in (left, right, diag):
            pl.semaphore_signal(barrier, inc=1, device_id=(nbr,),
                                device_id_type=pl.DeviceIdType.MESH)

        koff = my * S_SH
        wq16 = wq_ref[...].astype(BF16)
        wo16 = wo_ref[...].astype(BF16)

        def rdma(slot, c, target):
            return pltpu.make_async_remote_copy(
                src_ref=pbuf.at[c], dst_ref=rbuf.at[slot, c],
                send_sem=csend.at[slot, c], recv_sem=crecv.at[slot, c],
                device_id=(target,), device_id_type=pl.DeviceIdType.MESH)

        sends = [[None] * 3 for _ in range(NC)]
        first = True
        for b in range(B):
            k16 = [k_ref[b, :, h, :].astype(BF16) for h in range(Hq)]
            v16 = [v_ref[b, :, h, :].astype(BF16) for h in range(Hq)]
            for half in range(NCH):
                c = b * NCH + half
                r0 = half * SQ_C
                qi = lax.broadcasted_iota(jnp.int32, (SQ_C, S_SH), 0) + r0
                kig = lax.broadcasted_iota(jnp.int32, (SQ_C, S_SH), 1) + koff
                mask = (jnp.abs(qi - kig) <= 128) | (kig < 32) | (qi < 32)

                q_c = jnp.dot(
                    x_ref[b, r0:r0 + SQ_C, :].astype(BF16), wq16,
                    preferred_element_type=F32).astype(BF16)
                lcols = []
                for h in range(Hq):
                    qh = q_c[:, h * Dh:(h + 1) * Dh]
                    s = lax.dot_general(
                        qh, k16[h], (((1,), (1,)), ((), ())),
                        preferred_element_type=F32) * 0.125
                    w = jnp.where(mask, jnp.exp(s), 0.0)
                    pbuf[c, :SQ_C, h * Dh:(h + 1) * Dh] = jnp.dot(
                        w.astype(BF16), v16[h],
                        preferred_element_type=F32).astype(BF16)
                    lcols.append(jnp.sum(w, axis=1, keepdims=True))
                l_t = jnp.transpose(
                    jnp.concatenate(
                        lcols + [jnp.zeros((SQ_C, 8 - Hq), F32)],
                        axis=1))
                pbuf[c, SQ_C:, :] = jnp.concatenate(
                    [l_t, jnp.zeros((8, HD - SQ_C), F32)],
                    axis=1).astype(BF16)
                if first:
                    pl.semaphore_wait(barrier, 3)
                    first = False
                for slot, target in ((0, right), (1, left), (2, diag)):
                    sends[c][slot] = rdma(slot, c, target)
                    sends[c][slot].start()

        for c in range(NC):
            b, half = divmod(c, NCH)
            for slot in range(3):
                sends[c][slot].wait()
            tot = (pbuf[c].astype(F32) + rbuf[0, c].astype(F32)
                   + rbuf[1, c].astype(F32) + rbuf[2, c].astype(F32))
            ctx = tot[:SQ_C, :]
            l_c = jnp.transpose(tot[SQ_C:, :SQ_C])
            rcp = 1.0 / l_c
            parts = []
            for h in range(Hq):
                parts.append(ctx[:, h * Dh:(h + 1) * Dh] * rcp[:, h:h + 1])
            ctx_n = jnp.concatenate(parts, axis=1)
            out_ref[b, half * SQ_C:(half + 1) * SQ_C, :] = jnp.dot(
                ctx_n.astype(BF16), wo16, preferred_element_type=F32)

    return pl.pallas_call(
        body,
        out_shape=jax.ShapeDtypeStruct((B, Sq, D), jnp.float32),
        in_specs=[pl.BlockSpec(memory_space=pltpu.VMEM)] * 5,
        out_specs=pl.BlockSpec(memory_space=pltpu.VMEM),
        scratch_shapes=[
            pltpu.VMEM((NC, RC, HD), BF16),
            pltpu.VMEM((3, NC, RC, HD), BF16),
            pltpu.SemaphoreType.DMA((3, NC)),
            pltpu.SemaphoreType.DMA((3, NC)),
        ],
        compiler_params=pltpu.CompilerParams(collective_id=0),
    )(x, Wq, K_ext, V_ext, Wo)


# device time: 10243 ns/iter; 1.6235x vs baseline; 1.6235x over previous
import jax
import jax.numpy as jnp
from jax import lax
from jax.experimental import pallas as pl
from jax.experimental.pallas import tpu as pltpu

N_DEV = 4
B, Sq, Skv, Hq, Dh = 2, 256, 1024, 4, 64
HD = Hq * Dh
D = 512
S_SH = Skv // N_DEV
NCH = 2
SQ_C = Sq // NCH
NC = B * NCH
RC = SQ_C + 8
F32 = jnp.float32
BF16 = jnp.bfloat16


def kernel(x, Wq, K_ext, V_ext, Wo):
    Q16 = jnp.dot(x.reshape(B * Sq, D), Wq * 0.125,
                  preferred_element_type=F32).astype(BF16).reshape(B, Sq, HD)
    K16 = K_ext.reshape(B, S_SH, HD).astype(BF16)
    V16 = V_ext.reshape(B, S_SH, HD).astype(BF16)

    def body(q_ref, k_ref, v_ref, out_ref, pbuf, rbuf, csend, crecv):
        my = lax.axis_index("i")
        left = lax.rem(my + N_DEV - 1, N_DEV)
        right = lax.rem(my + 1, N_DEV)
        diag = lax.rem(my + 2, N_DEV)

        barrier = pltpu.get_barrier_semaphore()
        for nbr in (left, right, diag):
            pl.semaphore_signal(barrier, inc=1, device_id=(nbr,),
                                device_id_type=pl.DeviceIdType.MESH)

        koff = my * S_SH

        def rdma(slot, c, target):
            return pltpu.make_async_remote_copy(
                src_ref=pbuf.at[c], dst_ref=rbuf.at[slot, c],
                send_sem=csend.at[slot, c], recv_sem=crecv.at[slot, c],
                device_id=(target,), device_id_type=pl.DeviceIdType.MESH)

        sends = [[None] * 3 for _ in range(NC)]
        first = True
        for b in range(B):
            for half in range(NCH):
                c = b * NCH + half
                r0 = half * SQ_C
                qi = lax.broadcasted_iota(jnp.int32, (SQ_C, S_SH), 0) + r0
                kig = lax.broadcasted_iota(jnp.int32, (SQ_C, S_SH), 1) + koff
                mask = (jnp.abs(qi - kig) <= 128) | (kig < 32) | (qi < 32)

                lcols = []
                for h in range(Hq):
                    qh = q_ref[b, r0:r0 + SQ_C, h * Dh:(h + 1) * Dh]
                    kh = k_ref[b, :, h * Dh:(h + 1) * Dh]
                    s = lax.dot_general(
                        qh, kh, (((1,), (1,)), ((), ())),
                        preferred_element_type=F32)
                    w = jnp.where(mask, jnp.exp(s), 0.0)
                    vh = v_ref[b, :, h * Dh:(h + 1) * Dh]
                    pbuf[c, :SQ_C, h * Dh:(h + 1) * Dh] = jnp.dot(
                        w.astype(BF16), vh,
                        preferred_element_type=F32).astype(BF16)
                    lcols.append(jnp.sum(w, axis=1, keepdims=True))
                l_t = jnp.transpose(
                    jnp.concatenate(
                        lcols + [jnp.zeros((SQ_C, 8 - Hq), F32)],
                        axis=1))
                pbuf[c, SQ_C:, :] = jnp.concatenate(
                    [l_t, jnp.zeros((8, HD - SQ_C), F32)],
                    axis=1).astype(BF16)
                if first:
                    pl.semaphore_wait(barrier, 3)
                    first = False
                for slot, target in ((0, right), (1, left), (2, diag)):
                    sends[c][slot] = rdma(slot, c, target)
                    sends[c][slot].start()

        for c in range(NC):
            b, half = divmod(c, NCH)
            for slot in range(3):
                sends[c][slot].wait()
            tot = (pbuf[c].astype(F32) + rbuf[0, c].astype(F32)
                   + rbuf[1, c].astype(F32) + rbuf[2, c].astype(F32))
            ctx = tot[:SQ_C, :]
            l_c = jnp.transpose(tot[SQ_C:, :SQ_C])
            rcp = 1.0 / l_c
            parts = []
            for h in range(Hq):
                parts.append(ctx[:, h * Dh:(h + 1) * Dh] * rcp[:, h:h + 1])
            out_ref[b, half * SQ_C:(half + 1) * SQ_C, :] = jnp.concatenate(
                parts, axis=1).astype(BF16)

    ctx16 = pl.pallas_call(
        body,
        out_shape=jax.ShapeDtypeStruct((B, Sq, HD), BF16),
        in_specs=[pl.BlockSpec(memory_space=pltpu.VMEM)] * 3,
        out_specs=pl.BlockSpec(memory_space=pltpu.VMEM),
        scratch_shapes=[
            pltpu.VMEM((NC, RC, HD), BF16),
            pltpu.VMEM((3, NC, RC, HD), BF16),
            pltpu.SemaphoreType.DMA((3, NC)),
            pltpu.SemaphoreType.DMA((3, NC)),
        ],
        compiler_params=pltpu.CompilerParams(collective_id=0),
    )(Q16, K16, V16)

    out = jnp.dot(ctx16.reshape(B * Sq, HD), Wo.astype(BF16),
                  preferred_element_type=F32)
    return out.reshape(B, Sq, D)
